# baseline (speedup 1.0000x reference)
import functools
import math

import jax
import jax.numpy as jnp
from jax import lax
from jax.experimental import pallas as pl
from jax.experimental.pallas import tpu as pltpu
from jax.experimental.pallas import tpu_sc as plsc

N = 10000
E = 160000
LD = 256
VD = 64
B = 8
_MIN_LOGVAR = 2.0 * math.log(0.1)

_NBLK = 1000
_EBLK = 640
_HLD = LD // 2
_EH = E // 2

_F32 = jnp.float32
_BF16 = jnp.bfloat16


def _lnorm(x, g, b):
    m = jnp.mean(x, axis=-1, keepdims=True)
    d = x - m
    v = jnp.mean(d * d, axis=-1, keepdims=True)
    return d * lax.rsqrt(v + 1e-5) * g + b


def _dot(a, b):
    return jnp.dot(a, b, preferred_element_type=_F32)


def _pack_tc(x):
    xl, xr = x[:, :_HLD], x[:, _HLD:]
    lo = lax.shift_right_logical(
        lax.bitcast_convert_type(xl.astype(_BF16).astype(_F32), jnp.int32), 16)
    hi = jnp.bitwise_and(
        lax.bitcast_convert_type(xr.astype(_BF16).astype(_F32), jnp.int32),
        jnp.int32(-65536))
    return jnp.bitwise_or(hi, lo)


def _unpack_tc(p):
    lo = lax.bitcast_convert_type(lax.shift_left(p, 16), _F32)
    hi = lax.bitcast_convert_type(jnp.bitwise_and(p, jnp.int32(-65536)), _F32)
    return lo, hi


def _enc_body(x_ref, w1, b1, w2, b2, g, be, out_ref):
    pre = lax.dot_general(x_ref[...], w1[...], (((0,), (0,)), ((), ())),
                          preferred_element_type=_F32)
    h = jnp.maximum(pre + b1[...], 0.0)
    out = _lnorm(_dot(h.astype(_BF16), w2[...]) + b2[...], g[...], be[...])
    out_ref[...] = out.astype(out_ref.dtype)


def _enc_node_body(x_ref, w1, b1, w2, b2, g, be, wb, wc, h_ref, hb_ref, hc_ref):
    hid = jnp.maximum(_dot(x_ref[...], w1[...]) + b1[...], 0.0)
    h = _lnorm(_dot(hid, w2[...]) + b2[...], g[...], be[...])
    h_ref[...] = h
    hb_ref[...] = _pack_tc(_dot(h, wb[...]))
    hc_ref[...] = _pack_tc(_dot(h, wc[...]))


def _edge_body(e_ref, gb_ref, gc_ref, w1e, b1, w2, b2, g, be, enl_ref, enr_ref,
               enext_ref=None):
    e = e_ref[...]
    gbl, gbh = _unpack_tc(gb_ref[...])
    gcl, gch = _unpack_tc(gc_ref[...])
    gsum = jnp.concatenate([gbl + gcl, gbh + gch], axis=1)
    pre = _dot(e, w1e[...]) + gsum + b1[...]
    hid = jnp.maximum(pre, 0.0)
    enew = _lnorm(_dot(hid.astype(_BF16), w2[...]) + b2[...], g[...], be[...])
    enl_ref[...] = enew[:, :_HLD]
    enr_ref[...] = enew[:, _HLD:]
    if enext_ref is not None:
        enext_ref[...] = (e.astype(_F32) + enew).astype(_BF16)


def _edge1_body(xt_ref, gb_ref, gc_ref, ew1, eb1, ew2, eb2, eg, ebe,
                w1e, b1, w2, b2, g, be, enl_ref, enr_ref, enext_ref):
    pre0 = lax.dot_general(xt_ref[...], ew1[...], (((0,), (0,)), ((), ())),
                           preferred_element_type=_F32)
    h0 = jnp.maximum(pre0 + eb1[...], 0.0)
    e = _lnorm(_dot(h0.astype(_BF16), ew2[...]) + eb2[...], eg[...], ebe[...])
    gbl, gbh = _unpack_tc(gb_ref[...])
    gcl, gch = _unpack_tc(gc_ref[...])
    gsum = jnp.concatenate([gbl + gcl, gbh + gch], axis=1)
    pre = _dot(e.astype(_BF16), w1e[...]) + gsum + b1[...]
    hid = jnp.maximum(pre, 0.0)
    enew = _lnorm(_dot(hid.astype(_BF16), w2[...]) + b2[...], g[...], be[...])
    enl_ref[...] = enew[:, :_HLD]
    enr_ref[...] = enew[:, _HLD:]
    enext_ref[...] = (e + enew).astype(_BF16)


def _node1_body(h_ref, ala_ref, ara_ref, alb_ref, arb_ref, w1h, w1al, w1ar,
                b1, w2, b2, g, be, wb, wc, h2_ref, hb_ref, hc_ref):
    al = ala_ref[...] + alb_ref[...]
    ar = ara_ref[...] + arb_ref[...]
    hid = jnp.maximum(_dot(h_ref[...], w1h[...]) + _dot(al, w1al[...])
                      + _dot(ar, w1ar[...]) + b1[...], 0.0)
    hn = _lnorm(_dot(hid, w2[...]) + b2[...], g[...], be[...])
    h2 = h_ref[...] + hn
    h2_ref[...] = h2
    hb_ref[...] = _pack_tc(_dot(h2, wb[...]))
    hc_ref[...] = _pack_tc(_dot(h2, wc[...]))


def _node2_body(h_ref, ala_ref, ara_ref, alb_ref, arb_ref, bcol_ref, w1h,
                w1al, w1ar, b1, w2, b2, g, be, gw, gb0,
                h3_ref, gate_ref, gmax_ref):
    i = pl.program_id(0)
    al = ala_ref[...] + alb_ref[...]
    ar = ara_ref[...] + arb_ref[...]
    hid = jnp.maximum(_dot(h_ref[...], w1h[...]) + _dot(al, w1al[...])
                      + _dot(ar, w1ar[...]) + b1[...], 0.0)
    hn = _lnorm(_dot(hid, w2[...]) + b2[...], g[...], be[...])
    h3 = h_ref[...] + hn
    h3_ref[...] = h3
    gate = _dot(h3, gw[...]) + gb0[...]
    gate_ref[...] = gate
    onehot = bcol_ref[...] == lax.broadcasted_iota(jnp.int32, (_NBLK, B), 1)
    m = jnp.where(onehot, gate, -1e30)
    bmax = jnp.max(m, axis=0, keepdims=True)

    @pl.when(i == 0)
    def _():
        gmax_ref[...] = bmax

    @pl.when(i > 0)
    def _():
        gmax_ref[...] = jnp.maximum(gmax_ref[...], bmax)


def _pool_body(h3_ref, gate_ref, bcol_ref, gmax_ref, hgnum_ref, denom_ref):
    i = pl.program_id(0)
    onehot = (bcol_ref[...] == lax.broadcasted_iota(jnp.int32, (_NBLK, B), 1)
              ).astype(_F32)
    gm_row = jnp.sum(onehot * gmax_ref[...], axis=1, keepdims=True)
    ge = jnp.exp(gate_ref[...] - gm_row)
    geh = ge * h3_ref[...]
    hg = lax.dot_general(onehot, geh, (((0,), (0,)), ((), ())),
                         preferred_element_type=_F32)
    dn = lax.dot_general(onehot, ge, (((0,), (0,)), ((), ())),
                         preferred_element_type=_F32)

    @pl.when(i == 0)
    def _():
        hgnum_ref[...] = hg
        denom_ref[...] = dn

    @pl.when(i > 0)
    def _():
        hgnum_ref[...] = hgnum_ref[...] + hg
        denom_ref[...] = denom_ref[...] + dn


def _head_body(hgnum_ref, denom_ref, muw, mub, lvw, lvb, eps_ref,
               z_ref, mu_ref, lv_ref):
    hgraph = hgnum_ref[...] / (denom_ref[...] + 1e-16)
    mu = _dot(hgraph, muw[...]) + mub[...]
    lv = jnp.maximum(_dot(hgraph, lvw[...]) + lvb[...], _MIN_LOGVAR)
    mu_ref[...] = mu
    lv_ref[...] = lv
    z_ref[...] = mu + jnp.exp(0.5 * lv) * eps_ref[...]


def _full(shape):
    return pl.BlockSpec(shape, lambda i: (0,) * len(shape))


def _rows(blk, width):
    return pl.BlockSpec((blk, width), lambda i: (i, 0))


def _enc_edge(x_t, p):
    return pl.pallas_call(
        _enc_body,
        grid=(E // _EBLK,),
        in_specs=[pl.BlockSpec((8, _EBLK), lambda i: (0, i)),
                  _full((8, LD)), _full((1, LD)),
                  _full((LD, LD)), _full((1, LD)), _full((1, LD)),
                  _full((1, LD))],
        out_specs=_rows(_EBLK, LD),
        out_shape=jax.ShapeDtypeStruct((E, LD), _BF16),
    )(x_t, p['W1'].astype(_BF16), p['b1'][None, :],
      p['W2'].astype(_BF16), p['b2'][None, :],
      p['g'][None, :], p['be'][None, :])


def _enc_node(x8, w1pad, p, wb, wc):
    return pl.pallas_call(
        _enc_node_body,
        grid=(N // _NBLK,),
        in_specs=[_rows(_NBLK, 8), _full((8, LD)), _full((1, LD)),
                  _full((LD, LD)), _full((1, LD)), _full((1, LD)),
                  _full((1, LD)), _full((LD, LD)), _full((LD, LD))],
        out_specs=[_rows(_NBLK, LD), _rows(_NBLK, _HLD), _rows(_NBLK, _HLD)],
        out_shape=[jax.ShapeDtypeStruct((N, LD), _F32),
                   jax.ShapeDtypeStruct((N, _HLD), jnp.int32),
                   jax.ShapeDtypeStruct((N, _HLD), jnp.int32)],
    )(x8, w1pad, p['b1'][None, :], p['W2'], p['b2'][None, :],
      p['g'][None, :], p['be'][None, :], wb, wc)


def _edge_round1(xt, gb, gc, pe, w1e, p, half):
    hoff = half * (_EH // _EBLK)
    return pl.pallas_call(
        _edge1_body,
        grid=(_EH // _EBLK,),
        in_specs=[pl.BlockSpec((8, _EBLK), lambda i: (0, i + hoff)),
                  _rows(_EBLK, _HLD), _rows(_EBLK, _HLD),
                  _full((8, LD)), _full((1, LD)),
                  _full((LD, LD)), _full((1, LD)),
                  _full((1, LD)), _full((1, LD)),
                  _full((LD, LD)), _full((1, LD)),
                  _full((LD, LD)), _full((1, LD)),
                  _full((1, LD)), _full((1, LD))],
        out_specs=[_rows(_EBLK, _HLD), _rows(_EBLK, _HLD), _rows(_EBLK, LD)],
        out_shape=[jax.ShapeDtypeStruct((_EH, _HLD), _F32),
                   jax.ShapeDtypeStruct((_EH, _HLD), _F32),
                   jax.ShapeDtypeStruct((_EH, LD), _BF16)],
    )(xt, gb, gc,
      pe['W1'].astype(_BF16), pe['b1'][None, :],
      pe['W2'].astype(_BF16), pe['b2'][None, :],
      pe['g'][None, :], pe['be'][None, :],
      w1e.astype(_BF16), p['b1'][None, :],
      p['W2'].astype(_BF16), p['b2'][None, :],
      p['g'][None, :], p['be'][None, :])


def _edge_round2(e, gb, gc, w1e, p):
    return pl.pallas_call(
        (lambda e_r, gb_r, gc_r, w, b1, w2, b2, g, be, ol, orr:
         _edge_body(e_r, gb_r, gc_r, w, b1, w2, b2, g, be, ol, orr)),
        grid=(_EH // _EBLK,),
        in_specs=[_rows(_EBLK, LD), _rows(_EBLK, _HLD), _rows(_EBLK, _HLD),
                  _full((LD, LD)), _full((1, LD)),
                  _full((LD, LD)), _full((1, LD)),
                  _full((1, LD)), _full((1, LD))],
        out_specs=[_rows(_EBLK, _HLD), _rows(_EBLK, _HLD)],
        out_shape=[jax.ShapeDtypeStruct((_EH, _HLD), _F32)] * 2,
    )(e, gb, gc, w1e.astype(_BF16), p['b1'][None, :],
      p['W2'].astype(_BF16), p['b2'][None, :],
      p['g'][None, :], p['be'][None, :])


def _node_round1(h, aggs, w1h, w1al, w1ar, p, wb, wc):
    return pl.pallas_call(
        _node1_body,
        grid=(N // _NBLK,),
        in_specs=[_rows(_NBLK, LD)] + [_rows(_NBLK, _HLD)] * 4 +
                 [_full((LD, LD)), _full((_HLD, LD)), _full((_HLD, LD)),
                  _full((1, LD)), _full((LD, LD)), _full((1, LD)),
                  _full((1, LD)), _full((1, LD)), _full((LD, LD)),
                  _full((LD, LD))],
        out_specs=[_rows(_NBLK, LD), _rows(_NBLK, _HLD), _rows(_NBLK, _HLD)],
        out_shape=[jax.ShapeDtypeStruct((N, LD), _F32),
                   jax.ShapeDtypeStruct((N, _HLD), jnp.int32),
                   jax.ShapeDtypeStruct((N, _HLD), jnp.int32)],
    )(h, *aggs, w1h, w1al, w1ar, p['b1'][None, :], p['W2'], p['b2'][None, :],
      p['g'][None, :], p['be'][None, :], wb, wc)


def _node_round2(h, aggs, bcol, w1h, w1al, w1ar, p, gw, gb0):
    return pl.pallas_call(
        _node2_body,
        grid=(N // _NBLK,),
        in_specs=[_rows(_NBLK, LD)] + [_rows(_NBLK, _HLD)] * 4 +
                 [_rows(_NBLK, 1),
                  _full((LD, LD)), _full((_HLD, LD)), _full((_HLD, LD)),
                  _full((1, LD)), _full((LD, LD)), _full((1, LD)),
                  _full((1, LD)), _full((1, LD)), _full((LD, 1)),
                  _full((1, 1))],
        out_specs=[_rows(_NBLK, LD), _rows(_NBLK, 1), _full((1, B))],
        out_shape=[jax.ShapeDtypeStruct((N, LD), _F32),
                   jax.ShapeDtypeStruct((N, 1), _F32),
                   jax.ShapeDtypeStruct((1, B), _F32)],
    )(h, *aggs, bcol, w1h, w1al, w1ar, p['b1'][None, :], p['W2'],
      p['b2'][None, :], p['g'][None, :], p['be'][None, :], gw, gb0[None, :])


def _pool(h3, gate, bcol, gmax):
    return pl.pallas_call(
        _pool_body,
        grid=(N // _NBLK,),
        in_specs=[_rows(_NBLK, LD), _rows(_NBLK, 1), _rows(_NBLK, 1),
                  _full((1, B))],
        out_specs=[_full((B, LD)), _full((B, 1))],
        out_shape=[jax.ShapeDtypeStruct((B, LD), _F32),
                   jax.ShapeDtypeStruct((B, 1), _F32)],
    )(h3, gate, bcol, gmax)


def _head(hgnum, denom, muw, mub, lvw, lvb, eps):
    return pl.pallas_call(
        _head_body,
        grid=(1,),
        in_specs=[_full((B, LD)), _full((B, 1)), _full((LD, VD)),
                  _full((1, VD)), _full((LD, VD)), _full((1, VD)),
                  _full((B, VD))],
        out_specs=[_full((B, VD))] * 3,
        out_shape=[jax.ShapeDtypeStruct((B, VD), _F32)] * 3,
    )(hgnum, denom, muw, mub[None, :], lvw, lvb[None, :], eps)


_EPW = _EH // 16
_GCH = 128
_GRING = 3
_GGRP = 13
_GTAIL = _EPW - _GGRP * _GRING * _GCH

_SCH = 40
_SNC = _EPW // _SCH
_SGRP = (_SNC - 1) // 2
_SRT = 624
_STAIL = N - 16 * _SRT


@functools.lru_cache(maxsize=None)
def _sc_kernels():
    mesh = plsc.VectorSubcoreMesh(core_axis_name="c", subcore_axis_name="s")

    def make_gather(half):
        @functools.partial(
            pl.kernel,
            out_type=(jax.ShapeDtypeStruct((_EH, _HLD), jnp.int32),
                      jax.ShapeDtypeStruct((_EH, _HLD), jnp.int32)),
            mesh=mesh,
            scratch_types=[
                pltpu.VMEM((_EPW,), jnp.int32),
                pltpu.VMEM((_GCH, _HLD), jnp.int32),
                pltpu.VMEM((_GCH, _HLD), jnp.int32),
                pltpu.VMEM((_GCH, _HLD), jnp.int32),
            ] + [pltpu.SemaphoreType.DMA] * 6,
        )
        def sc_gather2(hb_h, hc_h, src_h, dst_h, gb_h, gc_h,
                       idxv, buf0, buf1, buf2, g0, g1, g2, w0, w1, w2):
            c = lax.axis_index("c")
            s = lax.axis_index("s")
            base = s * _EPW
            gbase = half * _EH + base
            bufs = (buf0, buf1, buf2)
            gs = (g0, g1, g2)
            ws = (w0, w1, w2)

            def run(tab_h, idx_h, out_h):
                pltpu.sync_copy(idx_h.at[pl.ds(gbase, _EPW)], idxv)
                for b in range(_GRING):
                    pltpu.async_copy(tab_h.at[idxv.at[pl.ds(b * _GCH, _GCH)]],
                                     bufs[b], gs[b])

                def group(g, carry):
                    off0 = g * (_GRING * _GCH)
                    for b in range(_GRING):
                        off = off0 + b * _GCH
                        pltpu.make_async_copy(
                            tab_h.at[idxv.at[pl.ds(off, _GCH)]], bufs[b],
                            gs[b]).wait()
                        pltpu.async_copy(
                            bufs[b], out_h.at[pl.ds(base + off, _GCH)], ws[b])

                    @pl.when(g < _GGRP - 1)
                    def _():
                        for b in range(_GRING):
                            off = off0 + b * _GCH
                            pltpu.make_async_copy(
                                bufs[b], out_h.at[pl.ds(base + off, _GCH)],
                                ws[b]).wait()
                            pltpu.async_copy(
                                tab_h.at[idxv.at[pl.ds(off + _GRING * _GCH,
                                                       _GCH)]],
                                bufs[b], gs[b])

                    return carry

                lax.fori_loop(0, _GGRP, group, 0)
                last0 = (_GGRP - 1) * _GRING * _GCH
                for b in range(_GRING):
                    pltpu.make_async_copy(
                        bufs[b],
                        out_h.at[pl.ds(base + last0 + b * _GCH, _GCH)],
                        ws[b]).wait()
                toff = _GGRP * _GRING * _GCH
                pltpu.async_copy(tab_h.at[idxv.at[pl.ds(toff, _GTAIL)]],
                                 buf0.at[pl.ds(0, _GTAIL)], gs[0]).wait()
                pltpu.sync_copy(buf0.at[pl.ds(0, _GTAIL)],
                                out_h.at[pl.ds(base + toff, _GTAIL)])

            @pl.when(c == 0)
            def _():
                run(hb_h, src_h, gb_h)

            @pl.when(c == 1)
            def _():
                run(hc_h, dst_h, gc_h)

        return sc_gather2

    @functools.partial(
        pl.kernel,
        out_type=(jax.ShapeDtypeStruct((N, _HLD), _F32),
                  jax.ShapeDtypeStruct((N, _HLD), _F32)),
        mesh=mesh,
        scratch_types=[
            pltpu.VMEM((_SNC, _SCH), jnp.int32),
            pltpu.VMEM((_SCH, _HLD), _F32),
            pltpu.VMEM((_SCH, _HLD), _F32),
            pltpu.VMEM_SHARED((N, _HLD), _F32),
        ] + [pltpu.SemaphoreType.DMA] * 4,
    )
    def sc_segsum(enl_h, enr_h, dstr_h, zeros_h, aggl_h, aggr_h,
                  idxv, d0, d1, accum, r0s, r1s, a0s, a1s):
        c = lax.axis_index("c")
        s = lax.axis_index("s")
        row0 = s * _SRT
        pltpu.sync_copy(zeros_h.at[pl.ds(row0, _SRT)],
                        accum.at[pl.ds(row0, _SRT)])

        @pl.when(s == 15)
        def _():
            pltpu.sync_copy(zeros_h.at[pl.ds(16 * _SRT, _STAIL)],
                            accum.at[pl.ds(16 * _SRT, _STAIL)])

        pltpu.sync_copy(dstr_h.at[s], idxv)
        plsc.subcore_barrier()
        ebase = s * _EPW
        dbufs = (d0, d1)
        rs = (r0s, r1s)
        ascat = (a0s, a1s)

        def run(en_h, agg_h):
            for b in range(2):
                pltpu.async_copy(en_h.at[pl.ds(ebase + b * _SCH, _SCH)],
                                 dbufs[b], rs[b])

            def group(g, carry):
                for b in range(2):
                    ch = 2 * g + b
                    off = ebase + ch * _SCH
                    pltpu.make_async_copy(en_h.at[pl.ds(off, _SCH)],
                                          dbufs[b], rs[b]).wait()
                    pltpu.async_copy(dbufs[b], accum.at[idxv.at[ch]],
                                     ascat[b], add=True)
                for b in range(2):
                    ch = 2 * g + b

                    @pl.when(ch + 2 < _SNC)
                    def _(b=b, ch=ch):
                        pltpu.make_async_copy(dbufs[b],
                                              accum.at[idxv.at[ch]],
                                              ascat[b]).wait()
                        pltpu.async_copy(
                            en_h.at[pl.ds(ebase + (ch + 2) * _SCH, _SCH)],
                            dbufs[b], rs[b])

                return carry

            lax.fori_loop(0, _SGRP, group, 0)
            lastc = _SNC - 1
            pltpu.make_async_copy(
                en_h.at[pl.ds(ebase + lastc * _SCH, _SCH)], d0, r0s).wait()
            pltpu.async_copy(d0, accum.at[idxv.at[lastc]], a0s, add=True)
            pltpu.make_async_copy(d0, accum.at[idxv.at[lastc]], a0s).wait()
            pltpu.make_async_copy(d1, accum.at[idxv.at[lastc - 1]],
                                  a1s).wait()
            plsc.subcore_barrier()
            pltpu.sync_copy(accum.at[pl.ds(row0, _SRT)],
                            agg_h.at[pl.ds(row0, _SRT)])

            @pl.when(s == 15)
            def _():
                pltpu.sync_copy(accum.at[pl.ds(16 * _SRT, _STAIL)],
                                agg_h.at[pl.ds(16 * _SRT, _STAIL)])

        @pl.when(c == 0)
        def _():
            run(enl_h, aggl_h)

        @pl.when(c == 1)
        def _():
            run(enr_h, aggr_h)

    return make_gather(0), make_gather(1), sc_segsum


@jax.jit
def kernel(y, edge_index, edge_attr, batch, params):
    src = edge_index[0]
    dst = edge_index[1]
    bcol = batch.reshape(N, 1)
    dstr_a = dst[:_EH].reshape(16, _SNC, _SCH)
    dstr_b = dst[_EH:].reshape(16, _SNC, _SCH)
    zeros_h = jnp.zeros((N, _HLD), _F32)
    eps = jax.random.normal(jax.random.key(42), (B, VD), _F32)

    ne = params['ne']
    y8 = jnp.pad(y, ((0, 0), (0, 8 - y.shape[1])))
    ne_w1 = jnp.pad(ne['W1'], ((0, 8 - ne['W1'].shape[0]), (0, 0)))

    mp0, mp1 = params['mp']
    e_w1_0 = mp0['edge']['W1']
    w1e0, wb0, wc0 = e_w1_0[:LD], e_w1_0[LD:2 * LD], e_w1_0[2 * LD:]
    e_w1_1 = mp1['edge']['W1']
    w1e1, wb1, wc1 = e_w1_1[:LD], e_w1_1[LD:2 * LD], e_w1_1[2 * LD:]
    n_w1_0 = mp0['node']['W1']
    w1h0, w1a0l, w1a0r = (n_w1_0[:LD], n_w1_0[LD:LD + _HLD],
                          n_w1_0[LD + _HLD:])
    n_w1_1 = mp1['node']['W1']
    w1h1, w1a1l, w1a1r = (n_w1_1[:LD], n_w1_1[LD:LD + _HLD],
                          n_w1_1[LD + _HLD:])

    _gather_a, _gather_b, _sc_segsum = _sc_kernels()
    xt = edge_attr.T.astype(_BF16)

    h1, hb1, hc1 = _enc_node(y8, ne_w1, ne, wb0, wc0)

    gbA1, gcA1 = _gather_a(hb1, hc1, src, dst)
    gbB1, gcB1 = _gather_b(hb1, hc1, src, dst)
    enlA1, enrA1, eA1 = _edge_round1(xt, gbA1, gcA1, params['ee'], w1e0,
                                     mp0['edge'], half=0)
    alA1, arA1 = _sc_segsum(enlA1, enrA1, dstr_a, zeros_h)
    enlB1, enrB1, eB1 = _edge_round1(xt, gbB1, gcB1, params['ee'], w1e0,
                                     mp0['edge'], half=1)
    alB1, arB1 = _sc_segsum(enlB1, enrB1, dstr_b, zeros_h)
    h2, hb2, hc2 = _node_round1(h1, (alA1, arA1, alB1, arB1),
                                w1h0, w1a0l, w1a0r, mp0['node'], wb1, wc1)

    gbA2, gcA2 = _gather_a(hb2, hc2, src, dst)
    gbB2, gcB2 = _gather_b(hb2, hc2, src, dst)
    enlA2, enrA2 = _edge_round2(eA1, gbA2, gcA2, w1e1, mp1['edge'])
    alA2, arA2 = _sc_segsum(enlA2, enrA2, dstr_a, zeros_h)
    enlB2, enrB2 = _edge_round2(eB1, gbB2, gcB2, w1e1, mp1['edge'])
    alB2, arB2 = _sc_segsum(enlB2, enrB2, dstr_b, zeros_h)
    h3, gate, gmax = _node_round2(h2, (alA2, arA2, alB2, arB2), bcol,
                                  w1h1, w1a1l, w1a1r, mp1['node'],
                                  params['gate_W'], params['gate_b'])

    hgnum, denom = _pool(h3, gate, bcol, gmax)
    z, mu, logvar = _head(hgnum, denom, params['mu_W'], params['mu_b'],
                          params['lv_W'], params['lv_b'], eps)
    return (z, mu, logvar)


# speedup vs baseline: 1.0530x; 1.0530x over previous
import functools
import math

import jax
import jax.numpy as jnp
from jax import lax
from jax.experimental import pallas as pl
from jax.experimental.pallas import tpu as pltpu
from jax.experimental.pallas import tpu_sc as plsc

N = 10000
E = 160000
LD = 256
VD = 64
B = 8
_MIN_LOGVAR = 2.0 * math.log(0.1)

_NBLK = 1000
_EBLK = 1280
_HLD = LD // 2

_F32 = jnp.float32
_BF16 = jnp.bfloat16


def _lnorm(x, g, b):
    m = jnp.mean(x, axis=-1, keepdims=True)
    d = x - m
    v = jnp.mean(d * d, axis=-1, keepdims=True)
    return d * lax.rsqrt(v + 1e-5) * g + b


def _dot(a, b):
    return jnp.dot(a, b, preferred_element_type=_F32)


def _pack_tc(x):
    xl, xr = x[:, :_HLD], x[:, _HLD:]
    lo = lax.shift_right_logical(
        lax.bitcast_convert_type(xl.astype(_BF16).astype(_F32), jnp.int32), 16)
    hi = jnp.bitwise_and(
        lax.bitcast_convert_type(xr.astype(_BF16).astype(_F32), jnp.int32),
        jnp.int32(-65536))
    return jnp.bitwise_or(hi, lo)


def _unpack_tc(p):
    lo = lax.bitcast_convert_type(lax.shift_left(p, 16), _F32)
    hi = lax.bitcast_convert_type(jnp.bitwise_and(p, jnp.int32(-65536)), _F32)
    return lo, hi


def _enc_body(x_ref, w1, b1, w2, b2, g, be, out_ref):
    pre = lax.dot_general(x_ref[...], w1[...], (((0,), (0,)), ((), ())),
                          preferred_element_type=_F32)
    h = jnp.maximum(pre + b1[...], 0.0)
    out = _lnorm(_dot(h.astype(_BF16), w2[...]) + b2[...], g[...], be[...])
    out_ref[...] = out.astype(out_ref.dtype)


def _enc_node_body(x_ref, w1, b1, w2, b2, g, be, wb, wc, h_ref, hb_ref, hc_ref):
    hid = jnp.maximum(_dot(x_ref[...], w1[...]) + b1[...], 0.0)
    h = _lnorm(_dot(hid, w2[...]) + b2[...], g[...], be[...])
    h_ref[...] = h
    hb_ref[...] = _pack_tc(_dot(h, wb[...]))
    hc_ref[...] = _pack_tc(_dot(h, wc[...]))


def _edge_body(e_ref, gb_ref, gc_ref, w1e, b1, w2, b2, g, be, enl_ref, enr_ref,
               enext_ref=None):
    e = e_ref[...]
    gbl, gbh = _unpack_tc(gb_ref[...])
    gcl, gch = _unpack_tc(gc_ref[...])
    gsum = jnp.concatenate([gbl + gcl, gbh + gch], axis=1)
    pre = _dot(e, w1e[...]) + gsum + b1[...]
    hid = jnp.maximum(pre, 0.0)
    enew = _lnorm(_dot(hid.astype(_BF16), w2[...]) + b2[...], g[...], be[...])
    enl_ref[...] = enew[:, :_HLD]
    enr_ref[...] = enew[:, _HLD:]
    if enext_ref is not None:
        enext_ref[...] = (e.astype(_F32) + enew).astype(_BF16)


def _edge1_body(xt_ref, gb_ref, gc_ref, ew1, eb1, ew2, eb2, eg, ebe,
                w1e, b1, w2, b2, g, be, enl_ref, enr_ref, enext_ref):
    pre0 = lax.dot_general(xt_ref[...], ew1[...], (((0,), (0,)), ((), ())),
                           preferred_element_type=_F32)
    h0 = jnp.maximum(pre0 + eb1[...], 0.0)
    e = _lnorm(_dot(h0.astype(_BF16), ew2[...]) + eb2[...], eg[...], ebe[...])
    gbl, gbh = _unpack_tc(gb_ref[...])
    gcl, gch = _unpack_tc(gc_ref[...])
    gsum = jnp.concatenate([gbl + gcl, gbh + gch], axis=1)
    pre = _dot(e.astype(_BF16), w1e[...]) + gsum + b1[...]
    hid = jnp.maximum(pre, 0.0)
    enew = _lnorm(_dot(hid.astype(_BF16), w2[...]) + b2[...], g[...], be[...])
    enl_ref[...] = enew[:, :_HLD]
    enr_ref[...] = enew[:, _HLD:]
    enext_ref[...] = (e + enew).astype(_BF16)


def _node1_body(h_ref, al_ref, ar_ref, w1h, w1al, w1ar, b1, w2, b2, g, be,
                wb, wc, h2_ref, hb_ref, hc_ref):
    hid = jnp.maximum(_dot(h_ref[...], w1h[...]) + _dot(al_ref[...], w1al[...])
                      + _dot(ar_ref[...], w1ar[...]) + b1[...], 0.0)
    hn = _lnorm(_dot(hid, w2[...]) + b2[...], g[...], be[...])
    h2 = h_ref[...] + hn
    h2_ref[...] = h2
    hb_ref[...] = _pack_tc(_dot(h2, wb[...]))
    hc_ref[...] = _pack_tc(_dot(h2, wc[...]))


def _node2_body(h_ref, al_ref, ar_ref, bcol_ref, w1h, w1al, w1ar, b1, w2, b2,
                g, be, gw, gb0, h3_ref, gate_ref, gmax_ref):
    i = pl.program_id(0)
    hid = jnp.maximum(_dot(h_ref[...], w1h[...]) + _dot(al_ref[...], w1al[...])
                      + _dot(ar_ref[...], w1ar[...]) + b1[...], 0.0)
    hn = _lnorm(_dot(hid, w2[...]) + b2[...], g[...], be[...])
    h3 = h_ref[...] + hn
    h3_ref[...] = h3
    gate = _dot(h3, gw[...]) + gb0[...]
    gate_ref[...] = gate
    onehot = bcol_ref[...] == lax.broadcasted_iota(jnp.int32, (_NBLK, B), 1)
    m = jnp.where(onehot, gate, -1e30)
    bmax = jnp.max(m, axis=0, keepdims=True)

    @pl.when(i == 0)
    def _():
        gmax_ref[...] = bmax

    @pl.when(i > 0)
    def _():
        gmax_ref[...] = jnp.maximum(gmax_ref[...], bmax)


def _pool_body(h3_ref, gate_ref, bcol_ref, gmax_ref, hgnum_ref, denom_ref):
    i = pl.program_id(0)
    onehot = (bcol_ref[...] == lax.broadcasted_iota(jnp.int32, (_NBLK, B), 1)
              ).astype(_F32)
    gm_row = jnp.sum(onehot * gmax_ref[...], axis=1, keepdims=True)
    ge = jnp.exp(gate_ref[...] - gm_row)
    geh = ge * h3_ref[...]
    hg = lax.dot_general(onehot, geh, (((0,), (0,)), ((), ())),
                         preferred_element_type=_F32)
    dn = lax.dot_general(onehot, ge, (((0,), (0,)), ((), ())),
                         preferred_element_type=_F32)

    @pl.when(i == 0)
    def _():
        hgnum_ref[...] = hg
        denom_ref[...] = dn

    @pl.when(i > 0)
    def _():
        hgnum_ref[...] = hgnum_ref[...] + hg
        denom_ref[...] = denom_ref[...] + dn


def _head_body(hgnum_ref, denom_ref, muw, mub, lvw, lvb, eps_ref,
               z_ref, mu_ref, lv_ref):
    hgraph = hgnum_ref[...] / (denom_ref[...] + 1e-16)
    mu = _dot(hgraph, muw[...]) + mub[...]
    lv = jnp.maximum(_dot(hgraph, lvw[...]) + lvb[...], _MIN_LOGVAR)
    mu_ref[...] = mu
    lv_ref[...] = lv
    z_ref[...] = mu + jnp.exp(0.5 * lv) * eps_ref[...]


def _full(shape):
    return pl.BlockSpec(shape, lambda i: (0,) * len(shape))


def _rows(blk, width):
    return pl.BlockSpec((blk, width), lambda i: (i, 0))


def _enc_edge(x_t, p):
    return pl.pallas_call(
        _enc_body,
        grid=(E // _EBLK,),
        in_specs=[pl.BlockSpec((8, _EBLK), lambda i: (0, i)),
                  _full((8, LD)), _full((1, LD)),
                  _full((LD, LD)), _full((1, LD)), _full((1, LD)),
                  _full((1, LD))],
        out_specs=_rows(_EBLK, LD),
        out_shape=jax.ShapeDtypeStruct((E, LD), _BF16),
    )(x_t, p['W1'].astype(_BF16), p['b1'][None, :],
      p['W2'].astype(_BF16), p['b2'][None, :],
      p['g'][None, :], p['be'][None, :])


def _enc_node(x8, w1pad, p, wb, wc):
    return pl.pallas_call(
        _enc_node_body,
        grid=(N // _NBLK,),
        in_specs=[_rows(_NBLK, 8), _full((8, LD)), _full((1, LD)),
                  _full((LD, LD)), _full((1, LD)), _full((1, LD)),
                  _full((1, LD)), _full((LD, LD)), _full((LD, LD))],
        out_specs=[_rows(_NBLK, LD), _rows(_NBLK, _HLD), _rows(_NBLK, _HLD)],
        out_shape=[jax.ShapeDtypeStruct((N, LD), _F32),
                   jax.ShapeDtypeStruct((N, _HLD), jnp.int32),
                   jax.ShapeDtypeStruct((N, _HLD), jnp.int32)],
    )(x8, w1pad, p['b1'][None, :], p['W2'], p['b2'][None, :],
      p['g'][None, :], p['be'][None, :], wb, wc)


def _edge_round1(xt, gb, gc, pe, w1e, p):
    return pl.pallas_call(
        _edge1_body,
        grid=(E // _EBLK,),
        in_specs=[pl.BlockSpec((8, _EBLK), lambda i: (0, i)),
                  _rows(_EBLK, _HLD), _rows(_EBLK, _HLD),
                  _full((8, LD)), _full((1, LD)),
                  _full((LD, LD)), _full((1, LD)),
                  _full((1, LD)), _full((1, LD)),
                  _full((LD, LD)), _full((1, LD)),
                  _full((LD, LD)), _full((1, LD)),
                  _full((1, LD)), _full((1, LD))],
        out_specs=[_rows(_EBLK, _HLD), _rows(_EBLK, _HLD), _rows(_EBLK, LD)],
        out_shape=[jax.ShapeDtypeStruct((E, _HLD), _F32),
                   jax.ShapeDtypeStruct((E, _HLD), _F32),
                   jax.ShapeDtypeStruct((E, LD), _BF16)],
    )(xt, gb, gc,
      pe['W1'].astype(_BF16), pe['b1'][None, :],
      pe['W2'].astype(_BF16), pe['b2'][None, :],
      pe['g'][None, :], pe['be'][None, :],
      w1e.astype(_BF16), p['b1'][None, :],
      p['W2'].astype(_BF16), p['b2'][None, :],
      p['g'][None, :], p['be'][None, :])


def _edge_round2(e, gb, gc, w1e, p):
    return pl.pallas_call(
        (lambda e_r, gb_r, gc_r, w, b1, w2, b2, g, be, ol, orr:
         _edge_body(e_r, gb_r, gc_r, w, b1, w2, b2, g, be, ol, orr)),
        grid=(E // _EBLK,),
        in_specs=[_rows(_EBLK, LD), _rows(_EBLK, _HLD), _rows(_EBLK, _HLD),
                  _full((LD, LD)), _full((1, LD)),
                  _full((LD, LD)), _full((1, LD)),
                  _full((1, LD)), _full((1, LD))],
        out_specs=[_rows(_EBLK, _HLD), _rows(_EBLK, _HLD)],
        out_shape=[jax.ShapeDtypeStruct((E, _HLD), _F32)] * 2,
    )(e, gb, gc, w1e.astype(_BF16), p['b1'][None, :],
      p['W2'].astype(_BF16), p['b2'][None, :],
      p['g'][None, :], p['be'][None, :])


def _node_round1(h, al, ar, w1h, w1al, w1ar, p, wb, wc):
    return pl.pallas_call(
        _node1_body,
        grid=(N // _NBLK,),
        in_specs=[_rows(_NBLK, LD), _rows(_NBLK, _HLD), _rows(_NBLK, _HLD),
                  _full((LD, LD)), _full((_HLD, LD)), _full((_HLD, LD)),
                  _full((1, LD)), _full((LD, LD)), _full((1, LD)),
                  _full((1, LD)), _full((1, LD)), _full((LD, LD)),
                  _full((LD, LD))],
        out_specs=[_rows(_NBLK, LD), _rows(_NBLK, _HLD), _rows(_NBLK, _HLD)],
        out_shape=[jax.ShapeDtypeStruct((N, LD), _F32),
                   jax.ShapeDtypeStruct((N, _HLD), jnp.int32),
                   jax.ShapeDtypeStruct((N, _HLD), jnp.int32)],
    )(h, al, ar, w1h, w1al, w1ar, p['b1'][None, :], p['W2'], p['b2'][None, :],
      p['g'][None, :], p['be'][None, :], wb, wc)


def _node_round2(h, al, ar, bcol, w1h, w1al, w1ar, p, gw, gb0):
    return pl.pallas_call(
        _node2_body,
        grid=(N // _NBLK,),
        in_specs=[_rows(_NBLK, LD), _rows(_NBLK, _HLD), _rows(_NBLK, _HLD),
                  _rows(_NBLK, 1),
                  _full((LD, LD)), _full((_HLD, LD)), _full((_HLD, LD)),
                  _full((1, LD)), _full((LD, LD)), _full((1, LD)),
                  _full((1, LD)), _full((1, LD)), _full((LD, 1)),
                  _full((1, 1))],
        out_specs=[_rows(_NBLK, LD), _rows(_NBLK, 1), _full((1, B))],
        out_shape=[jax.ShapeDtypeStruct((N, LD), _F32),
                   jax.ShapeDtypeStruct((N, 1), _F32),
                   jax.ShapeDtypeStruct((1, B), _F32)],
    )(h, al, ar, bcol, w1h, w1al, w1ar, p['b1'][None, :], p['W2'],
      p['b2'][None, :], p['g'][None, :], p['be'][None, :], gw, gb0[None, :])


def _pool(h3, gate, bcol, gmax):
    return pl.pallas_call(
        _pool_body,
        grid=(N // _NBLK,),
        in_specs=[_rows(_NBLK, LD), _rows(_NBLK, 1), _rows(_NBLK, 1),
                  _full((1, B))],
        out_specs=[_full((B, LD)), _full((B, 1))],
        out_shape=[jax.ShapeDtypeStruct((B, LD), _F32),
                   jax.ShapeDtypeStruct((B, 1), _F32)],
    )(h3, gate, bcol, gmax)


def _head(hgnum, denom, muw, mub, lvw, lvb, eps):
    return pl.pallas_call(
        _head_body,
        grid=(1,),
        in_specs=[_full((B, LD)), _full((B, 1)), _full((LD, VD)),
                  _full((1, VD)), _full((LD, VD)), _full((1, VD)),
                  _full((B, VD))],
        out_specs=[_full((B, VD))] * 3,
        out_shape=[jax.ShapeDtypeStruct((B, VD), _F32)] * 3,
    )(hgnum, denom, muw, mub[None, :], lvw, lvb[None, :], eps)


_EPW = E // 16
_GCH = 128
_GRING = 3
_GGRP = 26
_GTAIL = _EPW - _GGRP * _GRING * _GCH

_SCH = 80
_SNC = _EPW // _SCH
_SGRP = (_SNC - 1) // 2
_SRT = 624
_STAIL = N - 16 * _SRT


@functools.lru_cache(maxsize=None)
def _sc_kernels():
    mesh = plsc.VectorSubcoreMesh(core_axis_name="c", subcore_axis_name="s")

    @functools.partial(
        pl.kernel,
        out_type=(jax.ShapeDtypeStruct((E, _HLD), jnp.int32),
                  jax.ShapeDtypeStruct((E, _HLD), jnp.int32)),
        mesh=mesh,
        scratch_types=[
            pltpu.VMEM((_EPW,), jnp.int32),
            pltpu.VMEM((_GCH, _HLD), jnp.int32),
            pltpu.VMEM((_GCH, _HLD), jnp.int32),
            pltpu.VMEM((_GCH, _HLD), jnp.int32),
        ] + [pltpu.SemaphoreType.DMA] * 6,
    )
    def sc_gather2(hb_h, hc_h, src_h, dst_h, gb_h, gc_h,
                   idxv, buf0, buf1, buf2, g0, g1, g2, w0, w1, w2):
        c = lax.axis_index("c")
        s = lax.axis_index("s")
        base = s * _EPW
        bufs = (buf0, buf1, buf2)
        gs = (g0, g1, g2)
        ws = (w0, w1, w2)

        def run(tab_h, idx_h, out_h):
            pltpu.sync_copy(idx_h.at[pl.ds(base, _EPW)], idxv)
            for b in range(_GRING):
                pltpu.async_copy(tab_h.at[idxv.at[pl.ds(b * _GCH, _GCH)]],
                                 bufs[b], gs[b])

            def group(g, carry):
                off0 = g * (_GRING * _GCH)
                for b in range(_GRING):
                    off = off0 + b * _GCH
                    pltpu.make_async_copy(
                        tab_h.at[idxv.at[pl.ds(off, _GCH)]], bufs[b],
                        gs[b]).wait()
                    pltpu.async_copy(bufs[b],
                                     out_h.at[pl.ds(base + off, _GCH)], ws[b])

                @pl.when(g < _GGRP - 1)
                def _():
                    for b in range(_GRING):
                        off = off0 + b * _GCH
                        pltpu.make_async_copy(
                            bufs[b], out_h.at[pl.ds(base + off, _GCH)],
                            ws[b]).wait()
                        pltpu.async_copy(
                            tab_h.at[idxv.at[pl.ds(off + _GRING * _GCH,
                                                   _GCH)]],
                            bufs[b], gs[b])

                return carry

            lax.fori_loop(0, _GGRP, group, 0)
            last0 = (_GGRP - 1) * _GRING * _GCH
            for b in range(_GRING):
                pltpu.make_async_copy(
                    bufs[b], out_h.at[pl.ds(base + last0 + b * _GCH, _GCH)],
                    ws[b]).wait()
            toff = _GGRP * _GRING * _GCH
            pltpu.async_copy(tab_h.at[idxv.at[pl.ds(toff, _GTAIL)]],
                             buf0.at[pl.ds(0, _GTAIL)], gs[0]).wait()
            pltpu.sync_copy(buf0.at[pl.ds(0, _GTAIL)],
                            out_h.at[pl.ds(base + toff, _GTAIL)])

        @pl.when(c == 0)
        def _():
            run(hb_h, src_h, gb_h)

        @pl.when(c == 1)
        def _():
            run(hc_h, dst_h, gc_h)

    @functools.partial(
        pl.kernel,
        out_type=(jax.ShapeDtypeStruct((N, _HLD), _F32),
                  jax.ShapeDtypeStruct((N, _HLD), _F32)),
        mesh=mesh,
        scratch_types=[
            pltpu.VMEM((_SNC, _SCH), jnp.int32),
            pltpu.VMEM((_SCH, _HLD), _F32),
            pltpu.VMEM((_SCH, _HLD), _F32),
            pltpu.VMEM_SHARED((N, _HLD), _F32),
        ] + [pltpu.SemaphoreType.DMA] * 4,
    )
    def sc_segsum(enl_h, enr_h, dstr_h, zeros_h, aggl_h, aggr_h,
                  idxv, d0, d1, accum, r0s, r1s, a0s, a1s):
        c = lax.axis_index("c")
        s = lax.axis_index("s")
        row0 = s * _SRT
        pltpu.sync_copy(zeros_h.at[pl.ds(row0, _SRT)],
                        accum.at[pl.ds(row0, _SRT)])

        @pl.when(s == 15)
        def _():
            pltpu.sync_copy(zeros_h.at[pl.ds(16 * _SRT, _STAIL)],
                            accum.at[pl.ds(16 * _SRT, _STAIL)])

        pltpu.sync_copy(dstr_h.at[s], idxv)
        plsc.subcore_barrier()
        ebase = s * _EPW
        dbufs = (d0, d1)
        rs = (r0s, r1s)
        ascat = (a0s, a1s)

        def run(en_h, agg_h):
            for b in range(2):
                pltpu.async_copy(en_h.at[pl.ds(ebase + b * _SCH, _SCH)],
                                 dbufs[b], rs[b])

            def group(g, carry):
                for b in range(2):
                    ch = 2 * g + b
                    off = ebase + ch * _SCH
                    pltpu.make_async_copy(en_h.at[pl.ds(off, _SCH)],
                                          dbufs[b], rs[b]).wait()
                    pltpu.async_copy(dbufs[b], accum.at[idxv.at[ch]],
                                     ascat[b], add=True)
                for b in range(2):
                    ch = 2 * g + b

                    @pl.when(ch + 2 < _SNC)
                    def _(b=b, ch=ch):
                        pltpu.make_async_copy(dbufs[b],
                                              accum.at[idxv.at[ch]],
                                              ascat[b]).wait()
                        pltpu.async_copy(
                            en_h.at[pl.ds(ebase + (ch + 2) * _SCH, _SCH)],
                            dbufs[b], rs[b])

                return carry

            lax.fori_loop(0, _SGRP, group, 0)
            lastc = _SNC - 1
            pltpu.make_async_copy(
                en_h.at[pl.ds(ebase + lastc * _SCH, _SCH)], d0, r0s).wait()
            pltpu.async_copy(d0, accum.at[idxv.at[lastc]], a0s, add=True)
            pltpu.make_async_copy(d0, accum.at[idxv.at[lastc]], a0s).wait()
            pltpu.make_async_copy(d1, accum.at[idxv.at[lastc - 1]],
                                  a1s).wait()
            plsc.subcore_barrier()
            pltpu.sync_copy(accum.at[pl.ds(row0, _SRT)],
                            agg_h.at[pl.ds(row0, _SRT)])

            @pl.when(s == 15)
            def _():
                pltpu.sync_copy(accum.at[pl.ds(16 * _SRT, _STAIL)],
                                agg_h.at[pl.ds(16 * _SRT, _STAIL)])

        @pl.when(c == 0)
        def _():
            run(enl_h, aggl_h)

        @pl.when(c == 1)
        def _():
            run(enr_h, aggr_h)

    return sc_gather2, sc_segsum


@jax.jit
def kernel(y, edge_index, edge_attr, batch, params):
    src = edge_index[0]
    dst = edge_index[1]
    bcol = batch.reshape(N, 1)
    dstr = dst.reshape(16, _SNC, _SCH)
    zeros_h = jnp.zeros((N, _HLD), _F32)
    eps = jax.random.normal(jax.random.key(42), (B, VD), _F32)

    ne = params['ne']
    y8 = jnp.pad(y, ((0, 0), (0, 8 - y.shape[1])))
    ne_w1 = jnp.pad(ne['W1'], ((0, 8 - ne['W1'].shape[0]), (0, 0)))

    mp0, mp1 = params['mp']
    e_w1_0 = mp0['edge']['W1']
    w1e0, wb0, wc0 = e_w1_0[:LD], e_w1_0[LD:2 * LD], e_w1_0[2 * LD:]
    e_w1_1 = mp1['edge']['W1']
    w1e1, wb1, wc1 = e_w1_1[:LD], e_w1_1[LD:2 * LD], e_w1_1[2 * LD:]
    n_w1_0 = mp0['node']['W1']
    w1h0, w1a0l, w1a0r = (n_w1_0[:LD], n_w1_0[LD:LD + _HLD],
                          n_w1_0[LD + _HLD:])
    n_w1_1 = mp1['node']['W1']
    w1h1, w1a1l, w1a1r = (n_w1_1[:LD], n_w1_1[LD:LD + _HLD],
                          n_w1_1[LD + _HLD:])

    _sc_gather2, _sc_segsum = _sc_kernels()

    h1, hb1, hc1 = _enc_node(y8, ne_w1, ne, wb0, wc0)

    gb1, gc1 = _sc_gather2(hb1, hc1, src, dst)
    enl1, enr1, e1 = _edge_round1(edge_attr.T.astype(_BF16), gb1, gc1,
                                  params['ee'], w1e0, mp0['edge'])
    al1, ar1 = _sc_segsum(enl1, enr1, dstr, zeros_h)
    h2, hb2, hc2 = _node_round1(h1, al1, ar1, w1h0, w1a0l, w1a0r,
                                mp0['node'], wb1, wc1)

    gb2, gc2 = _sc_gather2(hb2, hc2, src, dst)
    enl2, enr2 = _edge_round2(e1, gb2, gc2, w1e1, mp1['edge'])
    al2, ar2 = _sc_segsum(enl2, enr2, dstr, zeros_h)
    h3, gate, gmax = _node_round2(h2, al2, ar2, bcol, w1h1, w1a1l, w1a1r,
                                  mp1['node'], params['gate_W'],
                                  params['gate_b'])

    hgnum, denom = _pool(h3, gate, bcol, gmax)
    z, mu, logvar = _head(hgnum, denom, params['mu_W'], params['mu_b'],
                          params['lv_W'], params['lv_b'], eps)
    return (z, mu, logvar)
